# baseline (speedup 1.0000x reference)
import functools

import jax
import jax.numpy as jnp
from jax import lax
from jax.experimental import pallas as pl
from jax.experimental.pallas import tpu as pltpu
from jax.experimental.pallas import tpu_sc as plsc

_GAMMA = 24.0
_L = 16


def kernel(entity_table, relation_table, positive_sample, negative_sample,
           q_entity, q_relation):
    B = positive_sample.shape[0]
    NNEG = negative_sample.shape[1]
    D = entity_table.shape[1]
    W = 2 * _L
    WC = D // W
    info = plsc.get_sparse_core_info()
    NW = info.num_cores * info.num_subcores
    BPW = B // NW
    RPC = 4
    NCH = BPW // RPC
    CR = RPC * NNEG

    ent_bf = entity_table.astype(jnp.bfloat16)
    rel_bf = relation_table.astype(jnp.bfloat16)

    mesh = plsc.VectorSubcoreMesh(core_axis_name="c", subcore_axis_name="s")

    @functools.partial(
        pl.kernel,
        out_type=(jax.ShapeDtypeStruct((B,), jnp.float32),
                  jax.ShapeDtypeStruct((B, NNEG), jnp.float32)),
        mesh=mesh,
        compiler_params=pltpu.CompilerParams(
            needs_layout_passes=False, use_tc_tiling_on_sc=False),
        scratch_types=[
            pltpu.VMEM((BPW,), jnp.int32),
            pltpu.VMEM((BPW,), jnp.int32),
            pltpu.VMEM((BPW,), jnp.int32),
            pltpu.VMEM((BPW, NNEG), jnp.int32),
            pltpu.VMEM((BPW, D), jnp.bfloat16),
            pltpu.VMEM((BPW, D), jnp.bfloat16),
            pltpu.VMEM((BPW, D), jnp.bfloat16),
            pltpu.VMEM((CR, D), jnp.bfloat16),
            pltpu.VMEM((CR, D), jnp.bfloat16),
            pltpu.VMEM((BPW,), jnp.float32),
            pltpu.VMEM((BPW, NNEG), jnp.float32),
            pltpu.SemaphoreType.DMA,
            pltpu.SemaphoreType.DMA,
            pltpu.SemaphoreType.DMA,
        ],
    )
    def _gqe(ent_hbm, rel_hbm, pos_hbm, neg_hbm, qe_hbm, qr_hbm,
             out_pos_hbm, out_neg_hbm,
             qe_idx, qr_idx, pos_idx, neg_idx, row_buf, r_rows, q_rows,
             nbuf0, nbuf1, out_pos, out_neg, sem, nsem0, nsem1):
        wid = lax.axis_index("s") * info.num_cores + lax.axis_index("c")
        base = wid * BPW
        iota = lax.iota(jnp.int32, _L)

        pltpu.sync_copy(qe_hbm.at[pl.ds(base, BPW)], qe_idx)
        pltpu.sync_copy(qr_hbm.at[pl.ds(base, BPW)], qr_idx)
        pltpu.sync_copy(pos_hbm.at[pl.ds(base, BPW)], pos_idx)
        pltpu.sync_copy(neg_hbm.at[pl.ds(base, BPW)], neg_idx)

        def fire_chunk(c, buf, nsem):
            for i in range(RPC):
                pltpu.async_copy(ent_hbm.at[neg_idx.at[c * RPC + i]],
                                 buf.at[pl.ds(i * NNEG, NNEG)], nsem)

        def drain_chunk(buf, nsem):
            pltpu.make_async_copy(ent_hbm.at[pl.ds(0, CR)], buf, nsem).wait()

        def row_l1(buf, r, qv):
            acc = jnp.abs(buf[r, pl.ds(0, W)] - qv[0])
            for cc in range(1, WC):
                acc = acc + jnp.abs(buf[r, pl.ds(cc * W, W)] - qv[cc])
            lo, hi = plsc.unpack(acc, format=plsc.PackFormat.INTERLEAVED)
            return jnp.sum(lo + hi, axis=0)

        def compute_chunk(c, buf):
            @pl.loop(0, RPC)
            def _(i):
                row = c * RPC + i
                qv = [q_rows[row, pl.ds(cc * W, W)] for cc in range(WC)]
                res = jnp.zeros((_L,), jnp.float32)
                for j in range(NNEG):
                    s = row_l1(buf, i * NNEG + j, qv)
                    res = jnp.where(iota == (j % _L), _GAMMA - s, res)
                    if j % _L == _L - 1:
                        out_neg[row, pl.ds((j // _L) * _L, _L)] = res

        fire_chunk(0, nbuf0, nsem0)
        c1 = pltpu.async_copy(ent_hbm.at[qe_idx], row_buf, sem)
        c2 = pltpu.async_copy(rel_hbm.at[qr_idx], r_rows, sem)
        c1.wait()
        c2.wait()

        @pl.loop(0, BPW)
        def _(r):
            for cc in range(WC):
                sl = pl.ds(cc * W, W)
                q_rows[r, sl] = row_buf[r, sl] + r_rows[r, sl]

        pltpu.async_copy(ent_hbm.at[pos_idx], row_buf, sem).wait()

        @pl.loop(0, BPW // _L)
        def _(g):
            res = jnp.zeros((_L,), jnp.float32)
            for k in range(_L):
                row = g * _L + k
                qv = [q_rows[row, pl.ds(cc * W, W)] for cc in range(WC)]
                s = row_l1(row_buf, row, qv)
                res = jnp.where(iota == k, _GAMMA - s, res)
            out_pos[pl.ds(g * _L, _L)] = res

        @pl.loop(0, NCH // 2)
        def _(t):
            c0 = 2 * t
            fire_chunk(c0 + 1, nbuf1, nsem1)
            drain_chunk(nbuf0, nsem0)
            compute_chunk(c0, nbuf0)

            @pl.when(c0 + 2 < NCH)
            def _():
                fire_chunk(c0 + 2, nbuf0, nsem0)

            drain_chunk(nbuf1, nsem1)
            compute_chunk(c0 + 1, nbuf1)

        pltpu.sync_copy(out_pos, out_pos_hbm.at[pl.ds(base, BPW)])
        pltpu.sync_copy(out_neg, out_neg_hbm.at[pl.ds(base, BPW)])

    return _gqe(ent_bf, rel_bf, positive_sample, negative_sample,
                q_entity, q_relation)


# speedup vs baseline: 1.2625x; 1.2625x over previous
import functools

import jax
import jax.numpy as jnp
from jax import lax
from jax.experimental import pallas as pl
from jax.experimental.pallas import tpu as pltpu
from jax.experimental.pallas import tpu_sc as plsc

_GAMMA = 24.0
_L = 16


def kernel(entity_table, relation_table, positive_sample, negative_sample,
           q_entity, q_relation):
    B = positive_sample.shape[0]
    NNEG = negative_sample.shape[1]
    D = entity_table.shape[1]
    info = plsc.get_sparse_core_info()
    NW = info.num_cores * info.num_subcores
    BPW = B // NW
    DC = D // _L
    RPC = 4
    NCH = BPW // RPC
    CR = RPC * NNEG

    mesh = plsc.VectorSubcoreMesh(core_axis_name="c", subcore_axis_name="s")

    @functools.partial(
        pl.kernel,
        out_type=(jax.ShapeDtypeStruct((B,), jnp.float32),
                  jax.ShapeDtypeStruct((B, NNEG), jnp.float32)),
        mesh=mesh,
        compiler_params=pltpu.CompilerParams(
            needs_layout_passes=False, use_tc_tiling_on_sc=False),
        scratch_types=[
            pltpu.VMEM((BPW,), jnp.int32),
            pltpu.VMEM((BPW,), jnp.int32),
            pltpu.VMEM((BPW,), jnp.int32),
            pltpu.VMEM((BPW, NNEG), jnp.int32),
            pltpu.VMEM((BPW, D), jnp.float32),
            pltpu.VMEM((BPW, D), jnp.float32),
            pltpu.VMEM((BPW, D), jnp.float32),
            pltpu.VMEM((CR, D), jnp.float32),
            pltpu.VMEM((CR, D), jnp.float32),
            pltpu.VMEM((BPW,), jnp.float32),
            pltpu.VMEM((BPW, NNEG), jnp.float32),
            pltpu.SemaphoreType.DMA,
            pltpu.SemaphoreType.DMA,
            pltpu.SemaphoreType.DMA,
        ],
    )
    def _gqe(ent_hbm, rel_hbm, pos_hbm, neg_hbm, qe_hbm, qr_hbm,
             out_pos_hbm, out_neg_hbm,
             qe_idx, qr_idx, pos_idx, neg_idx, q_rows, r_rows, pos_rows,
             nbuf0, nbuf1, out_pos, out_neg, sem, nsem0, nsem1):
        wid = lax.axis_index("s") * info.num_cores + lax.axis_index("c")
        base = wid * BPW
        iota = lax.iota(jnp.int32, _L)

        pltpu.sync_copy(qe_hbm.at[pl.ds(base, BPW)], qe_idx)
        pltpu.sync_copy(qr_hbm.at[pl.ds(base, BPW)], qr_idx)
        pltpu.sync_copy(pos_hbm.at[pl.ds(base, BPW)], pos_idx)
        pltpu.sync_copy(neg_hbm.at[pl.ds(base, BPW)], neg_idx)

        def fire_chunk(c, buf, nsem):
            for i in range(RPC):
                pltpu.async_copy(ent_hbm.at[neg_idx.at[c * RPC + i]],
                                 buf.at[pl.ds(i * NNEG, NNEG)], nsem)

        def drain_chunk(buf, nsem):
            pltpu.make_async_copy(ent_hbm.at[pl.ds(0, CR)], buf,
                                  nsem).wait()

        def compute_chunk(c, buf):
            @pl.loop(0, RPC)
            def _(i):
                row = c * RPC + i
                qv = [q_rows[row, pl.ds(cc * _L, _L)] for cc in range(DC)]
                res = jnp.zeros((_L,), jnp.float32)
                for j in range(NNEG):
                    r = i * NNEG + j
                    acc = jnp.abs(buf[r, pl.ds(0, _L)] - qv[0])
                    for cc in range(1, DC):
                        acc = acc + jnp.abs(buf[r, pl.ds(cc * _L, _L)] - qv[cc])
                    s = jnp.sum(acc, axis=0)
                    res = jnp.where(iota == (j % _L), _GAMMA - s, res)
                    if j % _L == _L - 1:
                        out_neg[row, pl.ds((j // _L) * _L, _L)] = res

        fire_chunk(0, nbuf0, nsem0)
        c1 = pltpu.async_copy(ent_hbm.at[qe_idx], q_rows, sem)
        c2 = pltpu.async_copy(rel_hbm.at[qr_idx], r_rows, sem)
        c3 = pltpu.async_copy(ent_hbm.at[pos_idx], pos_rows, sem)
        c1.wait()
        c2.wait()
        c3.wait()

        @pl.loop(0, BPW)
        def _(r):
            for c in range(DC):
                sl = pl.ds(c * _L, _L)
                q_rows[r, sl] = q_rows[r, sl] + r_rows[r, sl]

        @pl.loop(0, BPW // _L)
        def _(g):
            res = jnp.zeros((_L,), jnp.float32)
            for k in range(_L):
                row = g * _L + k
                acc = jnp.abs(pos_rows[row, pl.ds(0, _L)]
                              - q_rows[row, pl.ds(0, _L)])
                for cc in range(1, DC):
                    sl = pl.ds(cc * _L, _L)
                    acc = acc + jnp.abs(pos_rows[row, sl] - q_rows[row, sl])
                s = jnp.sum(acc, axis=0)
                res = jnp.where(iota == k, _GAMMA - s, res)
            out_pos[pl.ds(g * _L, _L)] = res

        @pl.loop(0, NCH // 2)
        def _(t):
            c0 = 2 * t
            fire_chunk(c0 + 1, nbuf1, nsem1)
            drain_chunk(nbuf0, nsem0)
            compute_chunk(c0, nbuf0)

            @pl.when(c0 + 2 < NCH)
            def _():
                fire_chunk(c0 + 2, nbuf0, nsem0)

            drain_chunk(nbuf1, nsem1)
            compute_chunk(c0 + 1, nbuf1)

        pltpu.sync_copy(out_pos, out_pos_hbm.at[pl.ds(base, BPW)])
        pltpu.sync_copy(out_neg, out_neg_hbm.at[pl.ds(base, BPW)])

    return _gqe(entity_table, relation_table, positive_sample,
                negative_sample, q_entity, q_relation)


# speedup vs baseline: 1.2681x; 1.0044x over previous
import functools

import jax
import jax.numpy as jnp
from jax import lax
from jax.experimental import pallas as pl
from jax.experimental.pallas import tpu as pltpu
from jax.experimental.pallas import tpu_sc as plsc

_GAMMA = 24.0
_L = 16


def kernel(entity_table, relation_table, positive_sample, negative_sample,
           q_entity, q_relation):
    B = positive_sample.shape[0]
    NNEG = negative_sample.shape[1]
    D = entity_table.shape[1]
    info = plsc.get_sparse_core_info()
    NW = info.num_cores * info.num_subcores
    BPW = B // NW
    DC = D // _L
    RPC = 4
    NCH = BPW // RPC
    CR = RPC * NNEG

    mesh = plsc.VectorSubcoreMesh(core_axis_name="c", subcore_axis_name="s")

    @functools.partial(
        pl.kernel,
        out_type=(jax.ShapeDtypeStruct((B,), jnp.float32),
                  jax.ShapeDtypeStruct((B, NNEG), jnp.float32)),
        mesh=mesh,
        compiler_params=pltpu.CompilerParams(
            needs_layout_passes=False, use_tc_tiling_on_sc=False),
        scratch_types=[
            pltpu.VMEM((BPW,), jnp.int32),
            pltpu.VMEM((BPW,), jnp.int32),
            pltpu.VMEM((BPW,), jnp.int32),
            pltpu.VMEM((BPW, NNEG), jnp.int32),
            pltpu.VMEM((BPW, D), jnp.float32),
            pltpu.VMEM((BPW, D), jnp.float32),
            pltpu.VMEM((BPW, D), jnp.float32),
            pltpu.VMEM((CR, D), jnp.float32),
            pltpu.VMEM((CR, D), jnp.float32),
            pltpu.VMEM((BPW,), jnp.float32),
            pltpu.VMEM((BPW, NNEG), jnp.float32),
            pltpu.SemaphoreType.DMA,
            pltpu.SemaphoreType.DMA((RPC,)),
            pltpu.SemaphoreType.DMA((RPC,)),
        ],
    )
    def _gqe(ent_hbm, rel_hbm, pos_hbm, neg_hbm, qe_hbm, qr_hbm,
             out_pos_hbm, out_neg_hbm,
             qe_idx, qr_idx, pos_idx, neg_idx, q_rows, r_rows, pos_rows,
             nbuf0, nbuf1, out_pos, out_neg, sem, nsem0, nsem1):
        wid = lax.axis_index("s") * info.num_cores + lax.axis_index("c")
        base = wid * BPW
        iota = lax.iota(jnp.int32, _L)

        pltpu.sync_copy(qe_hbm.at[pl.ds(base, BPW)], qe_idx)
        pltpu.sync_copy(qr_hbm.at[pl.ds(base, BPW)], qr_idx)
        pltpu.sync_copy(pos_hbm.at[pl.ds(base, BPW)], pos_idx)
        pltpu.sync_copy(neg_hbm.at[pl.ds(base, BPW)], neg_idx)

        def fire_chunk(c, buf, nsem):
            for i in range(RPC):
                pltpu.async_copy(ent_hbm.at[neg_idx.at[c * RPC + i]],
                                 buf.at[pl.ds(i * NNEG, NNEG)], nsem.at[i])

        def drain_quarter(buf, nsem, i):
            pltpu.make_async_copy(ent_hbm.at[pl.ds(0, NNEG)],
                                  buf.at[pl.ds(i * NNEG, NNEG)],
                                  nsem.at[i]).wait()

        def compute_chunk(c, buf, nsem):
            @pl.loop(0, RPC)
            def _(i):
                drain_quarter(buf, nsem, i)
                row = c * RPC + i
                qv = [q_rows[row, pl.ds(cc * _L, _L)] for cc in range(DC)]
                res = jnp.zeros((_L,), jnp.float32)
                for j in range(NNEG):
                    r = i * NNEG + j
                    acc = jnp.abs(buf[r, pl.ds(0, _L)] - qv[0])
                    for cc in range(1, DC):
                        acc = acc + jnp.abs(buf[r, pl.ds(cc * _L, _L)] - qv[cc])
                    s = jnp.sum(acc, axis=0)
                    res = jnp.where(iota == (j % _L), _GAMMA - s, res)
                    if j % _L == _L - 1:
                        out_neg[row, pl.ds((j // _L) * _L, _L)] = res

        fire_chunk(0, nbuf0, nsem0)
        c1 = pltpu.async_copy(ent_hbm.at[qe_idx], q_rows, sem)
        c2 = pltpu.async_copy(rel_hbm.at[qr_idx], r_rows, sem)
        c3 = pltpu.async_copy(ent_hbm.at[pos_idx], pos_rows, sem)
        c1.wait()
        c2.wait()
        c3.wait()

        @pl.loop(0, BPW)
        def _(r):
            for c in range(DC):
                sl = pl.ds(c * _L, _L)
                q_rows[r, sl] = q_rows[r, sl] + r_rows[r, sl]

        @pl.loop(0, BPW // _L)
        def _(g):
            res = jnp.zeros((_L,), jnp.float32)
            for k in range(_L):
                row = g * _L + k
                acc = jnp.abs(pos_rows[row, pl.ds(0, _L)]
                              - q_rows[row, pl.ds(0, _L)])
                for cc in range(1, DC):
                    sl = pl.ds(cc * _L, _L)
                    acc = acc + jnp.abs(pos_rows[row, sl] - q_rows[row, sl])
                s = jnp.sum(acc, axis=0)
                res = jnp.where(iota == k, _GAMMA - s, res)
            out_pos[pl.ds(g * _L, _L)] = res

        @pl.loop(0, NCH // 2)
        def _(t):
            c0 = 2 * t
            fire_chunk(c0 + 1, nbuf1, nsem1)
            compute_chunk(c0, nbuf0, nsem0)

            @pl.when(c0 + 2 < NCH)
            def _():
                fire_chunk(c0 + 2, nbuf0, nsem0)

            compute_chunk(c0 + 1, nbuf1, nsem1)

        pltpu.sync_copy(out_pos, out_pos_hbm.at[pl.ds(base, BPW)])
        pltpu.sync_copy(out_neg, out_neg_hbm.at[pl.ds(base, BPW)])

    return _gqe(entity_table, relation_table, positive_sample,
                negative_sample, q_entity, q_relation)
